# Initial kernel scaffold; baseline (speedup 1.0000x reference)
import jax
import jax.numpy as jnp
from jax.experimental import pallas as pl


def kernel(xyz, skinning_weight, node_xyz, node_quat):
    raise NotImplementedError("write your pallas kernel here")


# speedup vs baseline: 9.8220x; 9.8220x over previous
import functools

import jax
import jax.numpy as jnp
from jax.experimental import pallas as pl
from jax.experimental.pallas import tpu as pltpu

_K = 8
_SIGMA = 0.5
_BLK = 256


def _rot_table_kernel(quat_t_ref, nxyz_t_ref, tbl_ref):
    q = quat_t_ref[...]
    w, x, y, z = q[0:1], q[1:2], q[2:3], q[3:4]
    norm = jnp.sqrt(w * w + x * x + y * y + z * z)
    norm = jnp.maximum(norm, 1e-8)
    w, x, y, z = w / norm, x / norm, y / norm, z / norm
    r00 = 1.0 - 2.0 * (y * y + z * z)
    r01 = 2.0 * (x * y - z * w)
    r02 = 2.0 * (x * z + y * w)
    r10 = 2.0 * (x * y + z * w)
    r11 = 1.0 - 2.0 * (x * x + z * z)
    r12 = 2.0 * (y * z - x * w)
    r20 = 2.0 * (x * z - y * w)
    r21 = 2.0 * (y * z + x * w)
    r22 = 1.0 - 2.0 * (x * x + y * y)
    tbl_ref[0:3, :] = nxyz_t_ref[...]
    tbl_ref[3:12, :] = jnp.concatenate(
        [r00, r01, r02, r10, r11, r12, r20, r21, r22], axis=0)


def _main_kernel(xyz_ref, sw_ref, nxyz_t_ref, tbl_ref,
                 deformed_ref, idx_ref, skw_ref, *, blk, m):
    x = xyz_ref[...]
    nxyz_t = nxyz_t_ref[...]
    q_sq = jnp.sum(x * x, axis=1, keepdims=True)
    k_sq = jnp.sum(nxyz_t * nxyz_t, axis=0, keepdims=True)
    dot = jax.lax.dot_general(x, nxyz_t, (((1,), (0,)), ((), ())),
                              preferred_element_type=jnp.float32)
    d2 = (q_sq + k_sq) - 2.0 * dot

    iota = jax.lax.broadcasted_iota(jnp.int32, (blk, m), 1)
    inf = jnp.float32(jnp.inf)
    running = d2
    acc = jnp.zeros((blk, m), jnp.float32)
    sw = sw_ref[...]
    idx_cols = []
    logit_cols = []
    logit0 = None
    for k in range(_K):
        vm = jnp.min(running, axis=1, keepdims=True)
        cand = jnp.where(running == vm, iota, jnp.int32(m))
        idxf = jnp.min(cand, axis=1, keepdims=True)
        onehot = iota == idxf
        dist = jnp.sqrt(jnp.maximum(vm, 0.0))
        logit = -dist / _SIGMA + sw[:, k:k + 1]
        if k == 0:
            logit0 = logit
        e = jnp.exp(logit - logit0)
        acc = acc + jnp.where(onehot, e, 0.0)
        running = jnp.where(onehot, inf, running)
        idx_cols.append(idxf)
        logit_cols.append(logit)

    logits = jnp.concatenate(logit_cols, axis=1)
    mx = jnp.max(logits, axis=1, keepdims=True)
    ez = jnp.exp(logits - mx)
    z = jnp.sum(ez, axis=1, keepdims=True)
    skw_ref[...] = ez / z
    idx_ref[...] = jnp.concatenate(idx_cols, axis=1)

    scale = jnp.exp(logit0 - mx) / z
    g = jax.lax.dot_general(acc, tbl_ref[...], (((1,), (1,)), ((), ())),
                            preferred_element_type=jnp.float32)
    g = g * scale
    tbar = g[:, 0:3]
    x0, x1, x2 = x[:, 0:1], x[:, 1:2], x[:, 2:3]
    r0 = g[:, 3:4] * x0 + g[:, 4:5] * x1 + g[:, 5:6] * x2
    r1 = g[:, 6:7] * x0 + g[:, 7:8] * x1 + g[:, 8:9] * x2
    r2 = g[:, 9:10] * x0 + g[:, 10:11] * x1 + g[:, 11:12] * x2
    deformed_ref[...] = jnp.concatenate([r0, r1, r2], axis=1) + tbar


def kernel(xyz, skinning_weight, node_xyz, node_quat):
    n = xyz.shape[0]
    m = node_xyz.shape[0]
    nxyz_t = node_xyz.T
    nquat_t = node_quat.T
    tbl = pl.pallas_call(
        _rot_table_kernel,
        out_shape=jax.ShapeDtypeStruct((12, m), jnp.float32),
    )(nquat_t, nxyz_t)

    blk = _BLK
    grid = (n // blk,)
    deformed, idx, skw = pl.pallas_call(
        functools.partial(_main_kernel, blk=blk, m=m),
        grid=grid,
        in_specs=[
            pl.BlockSpec((blk, 3), lambda i: (i, 0)),
            pl.BlockSpec((blk, _K), lambda i: (i, 0)),
            pl.BlockSpec((3, m), lambda i: (0, 0)),
            pl.BlockSpec((12, m), lambda i: (0, 0)),
        ],
        out_specs=[
            pl.BlockSpec((blk, 3), lambda i: (i, 0)),
            pl.BlockSpec((blk, _K), lambda i: (i, 0)),
            pl.BlockSpec((blk, _K), lambda i: (i, 0)),
        ],
        out_shape=[
            jax.ShapeDtypeStruct((n, 3), jnp.float32),
            jax.ShapeDtypeStruct((n, _K), jnp.int32),
            jax.ShapeDtypeStruct((n, _K), jnp.float32),
        ],
        compiler_params=pltpu.CompilerParams(
            dimension_semantics=("arbitrary",),
        ),
    )(xyz, skinning_weight, nxyz_t, tbl)
    return deformed, idx, skw


# speedup vs baseline: 13.4684x; 1.3712x over previous
import functools

import jax
from jax import lax
import jax.numpy as jnp
from jax.experimental import pallas as pl
from jax.experimental.pallas import tpu as pltpu
from jax.experimental.pallas import tpu_sc as plsc

_K = 8
_SIGMA = 0.5
_BLK = 256
_NW = 32
_LANES = 16


def _rot_table_kernel(quat_t_ref, nxyz_t_ref, tbl_ref):
    q = quat_t_ref[...]
    w, x, y, z = q[0:1], q[1:2], q[2:3], q[3:4]
    norm = jnp.sqrt(w * w + x * x + y * y + z * z)
    norm = jnp.maximum(norm, 1e-8)
    w, x, y, z = w / norm, x / norm, y / norm, z / norm
    r00 = 1.0 - 2.0 * (y * y + z * z)
    r01 = 2.0 * (x * y - z * w)
    r02 = 2.0 * (x * z + y * w)
    r10 = 2.0 * (x * y + z * w)
    r11 = 1.0 - 2.0 * (x * x + z * z)
    r12 = 2.0 * (y * z - x * w)
    r20 = 2.0 * (x * z - y * w)
    r21 = 2.0 * (y * z + x * w)
    r22 = 1.0 - 2.0 * (x * x + y * y)
    tbl_ref[0:3, :] = nxyz_t_ref[...]
    tbl_ref[3:12, :] = jnp.concatenate(
        [r00, r01, r02, r10, r11, r12, r20, r21, r22], axis=0)


def _topk_kernel(xyz_ref, sw_ref, nxyz_t_ref, idx_ref, skw_ref, *, blk, m):
    x = xyz_ref[...]
    nxyz_t = nxyz_t_ref[...]
    q_sq = jnp.sum(x * x, axis=1, keepdims=True)
    k_sq = jnp.sum(nxyz_t * nxyz_t, axis=0, keepdims=True)
    dot = lax.dot_general(x, nxyz_t, (((1,), (0,)), ((), ())),
                          preferred_element_type=jnp.float32)
    d2 = (q_sq + k_sq) - 2.0 * dot

    iota = lax.broadcasted_iota(jnp.int32, (blk, m), 1)
    inf = jnp.float32(jnp.inf)
    running = d2
    sw = sw_ref[...]
    idx_cols = []
    logit_cols = []
    for k in range(_K):
        vm = jnp.min(running, axis=1, keepdims=True)
        cand = jnp.where(running == vm, iota, jnp.int32(m))
        idxi = jnp.min(cand, axis=1, keepdims=True)
        onehot = iota == idxi
        dist = jnp.sqrt(jnp.maximum(vm, 0.0))
        logit = -dist / _SIGMA + sw[:, k:k + 1]
        running = jnp.where(onehot, inf, running)
        idx_cols.append(idxi)
        logit_cols.append(logit)

    logits = jnp.concatenate(logit_cols, axis=1)
    mx = jnp.max(logits, axis=1, keepdims=True)
    ez = jnp.exp(logits - mx)
    z = jnp.sum(ez, axis=1, keepdims=True)
    skw_ref[...] = ez / z
    idx_ref[...] = jnp.concatenate(idx_cols, axis=1)


def _sc_blend_kernel(xyz_hbm, idx_hbm, skw_hbm, tbl_hbm, out_hbm,
                     xyz_v, idx_v, skw_v, tbl_v, out_v, *, per, m):
    wid = lax.axis_index("s") * 2 + lax.axis_index("c")
    base = wid * per
    pltpu.sync_copy(tbl_hbm, tbl_v)
    pltpu.sync_copy(xyz_hbm.at[:, pl.ds(base, per)], xyz_v)
    pltpu.sync_copy(idx_hbm.at[:, pl.ds(base, per)], idx_v)
    pltpu.sync_copy(skw_hbm.at[:, pl.ds(base, per)], skw_v)

    def body(g, carry):
        s = g * _LANES
        acc = [jnp.zeros((_LANES,), jnp.float32) for _ in range(12)]
        for kk in range(_K):
            idx = idx_v[kk, pl.ds(s, _LANES)]
            wk = skw_v[kk, pl.ds(s, _LANES)]
            for f in range(12):
                frow = jnp.full((_LANES,), f, jnp.int32)
                val = plsc.load_gather(tbl_v, [frow, idx])
                acc[f] = acc[f] + wk * val
        x0 = xyz_v[0, pl.ds(s, _LANES)]
        x1 = xyz_v[1, pl.ds(s, _LANES)]
        x2 = xyz_v[2, pl.ds(s, _LANES)]
        out_v[0, pl.ds(s, _LANES)] = acc[0] + acc[3] * x0 + acc[4] * x1 + acc[5] * x2
        out_v[1, pl.ds(s, _LANES)] = acc[1] + acc[6] * x0 + acc[7] * x1 + acc[8] * x2
        out_v[2, pl.ds(s, _LANES)] = acc[2] + acc[9] * x0 + acc[10] * x1 + acc[11] * x2
        return carry

    lax.fori_loop(0, per // _LANES, body, 0)
    pltpu.sync_copy(out_v, out_hbm.at[:, pl.ds(base, per)])


def kernel(xyz, skinning_weight, node_xyz, node_quat):
    n = xyz.shape[0]
    m = node_xyz.shape[0]
    nxyz_t = node_xyz.T
    nquat_t = node_quat.T
    tbl = pl.pallas_call(
        _rot_table_kernel,
        out_shape=jax.ShapeDtypeStruct((12, m), jnp.float32),
    )(nquat_t, nxyz_t)

    blk = _BLK
    idx, skw = pl.pallas_call(
        functools.partial(_topk_kernel, blk=blk, m=m),
        grid=(n // blk,),
        in_specs=[
            pl.BlockSpec((blk, 3), lambda i: (i, 0)),
            pl.BlockSpec((blk, _K), lambda i: (i, 0)),
            pl.BlockSpec((3, m), lambda i: (0, 0)),
        ],
        out_specs=[
            pl.BlockSpec((blk, _K), lambda i: (i, 0)),
            pl.BlockSpec((blk, _K), lambda i: (i, 0)),
        ],
        out_shape=[
            jax.ShapeDtypeStruct((n, _K), jnp.int32),
            jax.ShapeDtypeStruct((n, _K), jnp.float32),
        ],
        compiler_params=pltpu.CompilerParams(
            dimension_semantics=("arbitrary",),
        ),
    )(xyz, skinning_weight, nxyz_t)

    per = n // _NW
    mesh = plsc.VectorSubcoreMesh(core_axis_name="c", subcore_axis_name="s")
    sc_blend = functools.partial(
        pl.kernel,
        mesh=mesh,
        out_type=jax.ShapeDtypeStruct((3, n), jnp.float32),
        scratch_types=[
            pltpu.VMEM((3, per), jnp.float32),
            pltpu.VMEM((_K, per), jnp.int32),
            pltpu.VMEM((_K, per), jnp.float32),
            pltpu.VMEM((12, m), jnp.float32),
            pltpu.VMEM((3, per), jnp.float32),
        ],
        compiler_params=pltpu.CompilerParams(needs_layout_passes=False),
    )(functools.partial(_sc_blend_kernel, per=per, m=m))
    deformed_t = sc_blend(xyz.T, idx.T, skw.T, tbl)
    return deformed_t.T, idx, skw


# speedup vs baseline: 14.6002x; 1.0840x over previous
import functools

import jax
from jax import lax
import jax.numpy as jnp
from jax.experimental import pallas as pl
from jax.experimental.pallas import tpu as pltpu
from jax.experimental.pallas import tpu_sc as plsc

_K = 8
_SIGMA = 0.5
_BLK = 256
_NW = 32
_LANES = 16


def _rot_table_kernel(quat_t_ref, nxyz_t_ref, tbl_ref):
    q = quat_t_ref[...]
    w, x, y, z = q[0:1], q[1:2], q[2:3], q[3:4]
    norm = jnp.sqrt(w * w + x * x + y * y + z * z)
    norm = jnp.maximum(norm, 1e-8)
    w, x, y, z = w / norm, x / norm, y / norm, z / norm
    r00 = 1.0 - 2.0 * (y * y + z * z)
    r01 = 2.0 * (x * y - z * w)
    r02 = 2.0 * (x * z + y * w)
    r10 = 2.0 * (x * y + z * w)
    r11 = 1.0 - 2.0 * (x * x + z * z)
    r12 = 2.0 * (y * z - x * w)
    r20 = 2.0 * (x * z - y * w)
    r21 = 2.0 * (y * z + x * w)
    r22 = 1.0 - 2.0 * (x * x + y * y)
    tbl_ref[0:3, :] = nxyz_t_ref[...]
    tbl_ref[3:12, :] = jnp.concatenate(
        [r00, r01, r02, r10, r11, r12, r20, r21, r22], axis=0)


_C = 128


def _topk_kernel(xyz_ref, sw_ref, nxyz_t_ref, idx_ref, skw_ref, *, blk, m):
    nc = m // _C
    x = xyz_ref[...]
    nxyz_t = nxyz_t_ref[...]
    q_sq = jnp.sum(x * x, axis=1, keepdims=True)
    k_sq = jnp.sum(nxyz_t * nxyz_t, axis=0, keepdims=True)
    dot = lax.dot_general(x, nxyz_t, (((1,), (0,)), ((), ())),
                          preferred_element_type=jnp.float32)
    d2 = (q_sq + k_sq) - 2.0 * dot
    inf = jnp.float32(jnp.inf)

    chunks = [d2[:, c * _C:(c + 1) * _C] for c in range(nc)]
    cm = jnp.concatenate(
        [jnp.min(ch, axis=1, keepdims=True) for ch in chunks], axis=1)

    iota_c = lax.broadcasted_iota(jnp.int32, (blk, nc), 1)
    runc = cm
    cs = []
    for s in range(_K):
        vmc = jnp.min(runc, axis=1, keepdims=True)
        candc = jnp.where(runc == vmc, iota_c, jnp.int32(nc))
        ci = jnp.min(candc, axis=1, keepdims=True)
        runc = jnp.where(iota_c == ci, inf, runc)
        cs.append(ci)

    slots = []
    for s in range(_K):
        acc = jnp.zeros((blk, _C), jnp.float32)
        for c in range(nc):
            acc = jnp.where(cs[s] == c, chunks[c], acc)
        slots.append(acc)
    cand_buf = jnp.concatenate(slots, axis=1)

    iota_e = lax.broadcasted_iota(jnp.int32, (blk, _K * _C), 1)
    run_e = cand_buf
    sw = sw_ref[...]
    idx_cols = []
    logit_cols = []
    for k in range(_K):
        vm = jnp.min(run_e, axis=1, keepdims=True)
        cand = jnp.where(run_e == vm, iota_e, jnp.int32(_K * _C))
        li = jnp.min(cand, axis=1, keepdims=True)
        run_e = jnp.where(iota_e == li, inf, run_e)
        slot = lax.shift_right_logical(li, 7)
        lane = jnp.bitwise_and(li, jnp.int32(_C - 1))
        c_sel = jnp.zeros((blk, 1), jnp.int32)
        for s in range(_K):
            c_sel = jnp.where(slot == s, cs[s], c_sel)
        gidx = c_sel * _C + lane
        dist = jnp.sqrt(jnp.maximum(vm, 0.0))
        logit = -dist / _SIGMA + sw[:, k:k + 1]
        idx_cols.append(gidx)
        logit_cols.append(logit)

    logits = jnp.concatenate(logit_cols, axis=1)
    mx = jnp.max(logits, axis=1, keepdims=True)
    ez = jnp.exp(logits - mx)
    z = jnp.sum(ez, axis=1, keepdims=True)
    skw_ref[...] = ez / z
    idx_ref[...] = jnp.concatenate(idx_cols, axis=1)


def _sc_blend_kernel(xyz_hbm, idx_hbm, skw_hbm, tbl_hbm, out_hbm,
                     xyz_v, idx_v, skw_v, tbl_v, out_v, *, per, m):
    wid = lax.axis_index("s") * 2 + lax.axis_index("c")
    base = wid * per
    pltpu.sync_copy(tbl_hbm, tbl_v)
    pltpu.sync_copy(xyz_hbm.at[:, pl.ds(base, per)], xyz_v)
    pltpu.sync_copy(idx_hbm.at[:, pl.ds(base, per)], idx_v)
    pltpu.sync_copy(skw_hbm.at[:, pl.ds(base, per)], skw_v)

    def body(g, carry):
        s = g * _LANES
        acc = [jnp.zeros((_LANES,), jnp.float32) for _ in range(12)]
        for kk in range(_K):
            idx = idx_v[kk, pl.ds(s, _LANES)]
            wk = skw_v[kk, pl.ds(s, _LANES)]
            for f in range(12):
                frow = jnp.full((_LANES,), f, jnp.int32)
                val = plsc.load_gather(tbl_v, [frow, idx])
                acc[f] = acc[f] + wk * val
        x0 = xyz_v[0, pl.ds(s, _LANES)]
        x1 = xyz_v[1, pl.ds(s, _LANES)]
        x2 = xyz_v[2, pl.ds(s, _LANES)]
        out_v[0, pl.ds(s, _LANES)] = acc[0] + acc[3] * x0 + acc[4] * x1 + acc[5] * x2
        out_v[1, pl.ds(s, _LANES)] = acc[1] + acc[6] * x0 + acc[7] * x1 + acc[8] * x2
        out_v[2, pl.ds(s, _LANES)] = acc[2] + acc[9] * x0 + acc[10] * x1 + acc[11] * x2
        return carry

    lax.fori_loop(0, per // _LANES, body, 0)
    pltpu.sync_copy(out_v, out_hbm.at[:, pl.ds(base, per)])


def kernel(xyz, skinning_weight, node_xyz, node_quat):
    n = xyz.shape[0]
    m = node_xyz.shape[0]
    nxyz_t = node_xyz.T
    nquat_t = node_quat.T
    tbl = pl.pallas_call(
        _rot_table_kernel,
        out_shape=jax.ShapeDtypeStruct((12, m), jnp.float32),
    )(nquat_t, nxyz_t)

    blk = _BLK
    idx, skw = pl.pallas_call(
        functools.partial(_topk_kernel, blk=blk, m=m),
        grid=(n // blk,),
        in_specs=[
            pl.BlockSpec((blk, 3), lambda i: (i, 0)),
            pl.BlockSpec((blk, _K), lambda i: (i, 0)),
            pl.BlockSpec((3, m), lambda i: (0, 0)),
        ],
        out_specs=[
            pl.BlockSpec((blk, _K), lambda i: (i, 0)),
            pl.BlockSpec((blk, _K), lambda i: (i, 0)),
        ],
        out_shape=[
            jax.ShapeDtypeStruct((n, _K), jnp.int32),
            jax.ShapeDtypeStruct((n, _K), jnp.float32),
        ],
        compiler_params=pltpu.CompilerParams(
            dimension_semantics=("arbitrary",),
        ),
    )(xyz, skinning_weight, nxyz_t)

    per = n // _NW
    mesh = plsc.VectorSubcoreMesh(core_axis_name="c", subcore_axis_name="s")
    sc_blend = functools.partial(
        pl.kernel,
        mesh=mesh,
        out_type=jax.ShapeDtypeStruct((3, n), jnp.float32),
        scratch_types=[
            pltpu.VMEM((3, per), jnp.float32),
            pltpu.VMEM((_K, per), jnp.int32),
            pltpu.VMEM((_K, per), jnp.float32),
            pltpu.VMEM((12, m), jnp.float32),
            pltpu.VMEM((3, per), jnp.float32),
        ],
        compiler_params=pltpu.CompilerParams(needs_layout_passes=False),
    )(functools.partial(_sc_blend_kernel, per=per, m=m))
    deformed_t = sc_blend(xyz.T, idx.T, skw.T, tbl)
    return deformed_t.T, idx, skw
